# Initial kernel scaffold; baseline (speedup 1.0000x reference)
import jax
import jax.numpy as jnp
from jax.experimental import pallas as pl


def kernel(image, label_image, label_costs):
    raise NotImplementedError("write your pallas kernel here")


# speedup vs baseline: 2.0250x; 2.0250x over previous
import functools

import jax
import jax.numpy as jnp
from jax import lax
from jax.experimental import pallas as pl
from jax.experimental.pallas import tpu as pltpu
from jax.experimental.pallas import tpu_sc as plsc

H, W = 1024, 2048
CROP_H, CROP_W = 720, 1280
SAMPLES = 4
LABEL_COUNT = 19
NC, NS = 2, 16
NWORK = NC * NS
RPW = 23
WB = 1296
NVEC = WB // 16
HIST_W = LABEL_COUNT * 16

_mesh = plsc.VectorSubcoreMesh(core_axis_name="c", subcore_axis_name="s")
_params = pltpu.CompilerParams(
    use_tc_tiling_on_sc=False, needs_layout_passes=False)


def _pick(vec, iota, k):
    return jnp.max(jnp.where(iota == k, vec, 0))


@functools.partial(
    pl.kernel,
    out_type=jax.ShapeDtypeStruct((NWORK, SAMPLES * HIST_W), jnp.int32),
    mesh=_mesh,
    scratch_types=[
        pltpu.VMEM((16,), jnp.int32),
        pltpu.VMEM((RPW, WB), jnp.int32),
        pltpu.VMEM((SAMPLES * HIST_W,), jnp.int32),
    ],
    compiler_params=_params,
)
def _hist_kernel(label_hbm, offs_hbm, out_hbm, offs_v, buf_v, hist_v):
    w = lax.axis_index("s") * NC + lax.axis_index("c")
    iota = lax.iota(jnp.int32, 16)
    zeros = jnp.zeros((16,), jnp.int32)
    ones = jnp.ones((16,), jnp.int32)

    for k in range(SAMPLES * HIST_W // 16):
        hist_v[pl.ds(k * 16, 16)] = zeros

    pltpu.sync_copy(offs_hbm, offs_v)
    ov = offs_v[...]

    lo = jnp.minimum(RPW * w, CROP_H - RPW)
    r_begin = RPW * w

    for c in range(SAMPLES):
        top = _pick(ov, iota, c)
        left = _pick(ov, iota, SAMPLES + c)
        left_al = pl.multiple_of(jnp.minimum(left & -16, W - WB), 16)
        shift = left - left_al
        mask_first = iota >= shift
        mask_last = iota < shift
        base = c * HIST_W + iota

        pltpu.sync_copy(
            label_hbm.at[pl.ds(top + lo, RPW), pl.ds(left_al, WB)], buf_v
        )

        def body(i, carry, _base=base, _mf=mask_first, _ml=mask_last):
            rg = lo + i
            valid = jnp.logical_and(rg >= r_begin, rg < CROP_H)
            rmask = jnp.full((16,), valid)
            for j in range(NVEC):
                lv = buf_v[i, pl.ds(j * 16, 16)]
                if j == 0:
                    m = jnp.logical_and(rmask, _mf)
                elif j == NVEC - 1:
                    m = jnp.logical_and(rmask, _ml)
                else:
                    m = rmask
                plsc.addupdate_scatter(hist_v, [lv * 16 + _base], ones, mask=m)
            return carry

        lax.fori_loop(0, RPW, body, 0)

    pltpu.sync_copy(hist_v, out_hbm.at[w])


@functools.partial(
    pl.kernel,
    out_type=(
        jax.ShapeDtypeStruct((3, CROP_H, CROP_W), jnp.float32),
        jax.ShapeDtypeStruct((CROP_H, CROP_W), jnp.int32),
    ),
    mesh=_mesh,
    scratch_types=[
        pltpu.VMEM((16,), jnp.int32),
        pltpu.VMEM((RPW, WB), jnp.float32),
        pltpu.VMEM((RPW, CROP_W), jnp.float32),
        pltpu.VMEM((RPW, WB), jnp.int32),
        pltpu.VMEM((RPW, CROP_W), jnp.int32),
    ],
    compiler_params=_params,
)
def _crop_kernel(img_hbm, lab_hbm, sel_hbm, oimg_hbm, olab_hbm,
                 sel_v, ibuf, obuf, lbuf, olbuf):
    w = lax.axis_index("s") * NC + lax.axis_index("c")
    iota = lax.iota(jnp.int32, 16)

    pltpu.sync_copy(sel_hbm, sel_v)
    sv = sel_v[...]
    top = _pick(sv, iota, 0)
    left = _pick(sv, iota, 1)
    left_al = pl.multiple_of(jnp.minimum(left & -16, W - WB), 16)
    shift = left - left_al
    lo = jnp.minimum(RPW * w, CROP_H - RPW)

    def shift_rows(src, dst):
        def body(i, carry):
            rowv = jnp.full((16,), i)
            for j in range(CROP_W // 16):
                v = plsc.load_gather(src, [rowv, shift + (j * 16) + iota])
                dst[i, pl.ds(j * 16, 16)] = v
            return carry
        lax.fori_loop(0, RPW, body, 0)

    for ch in range(3):
        pltpu.sync_copy(
            img_hbm.at[ch, pl.ds(top + lo, RPW), pl.ds(left_al, WB)], ibuf
        )
        shift_rows(ibuf, obuf)
        pltpu.sync_copy(obuf, oimg_hbm.at[ch, pl.ds(lo, RPW), :])

    pltpu.sync_copy(
        lab_hbm.at[pl.ds(top + lo, RPW), pl.ds(left_al, WB)], lbuf
    )
    shift_rows(lbuf, olbuf)
    pltpu.sync_copy(olbuf, olab_hbm.at[pl.ds(lo, RPW), :])


def kernel(image, label_image, label_costs):
    label2d = label_image.reshape(H, W)

    base = jax.random.key(42)
    tops, lefts = [], []
    for i in range(SAMPLES):
        k = jax.random.fold_in(base, i)
        kt, kl = jax.random.split(k)
        tops.append(jax.random.randint(kt, (), 0, H - CROP_H + 1))
        lefts.append(jax.random.randint(kl, (), 0, W - CROP_W + 1))

    offs = jnp.zeros((16,), jnp.int32)
    for i in range(SAMPLES):
        offs = offs.at[i].set(tops[i]).at[SAMPLES + i].set(lefts[i])

    parts = _hist_kernel(label2d, offs)
    counts = parts.reshape(NWORK, SAMPLES, LABEL_COUNT, 16).sum(axis=(0, 3))

    norm_costs = label_costs / jnp.maximum(jnp.sum(jnp.abs(label_costs)), 1e-12)

    def cost_of(c):
        hist = jnp.zeros((256,), jnp.float32).at[:LABEL_COUNT].set(
            counts[c].astype(jnp.float32))
        dist = (hist / jnp.maximum(jnp.sum(jnp.abs(hist)), 1e-12))[:LABEL_COUNT]
        return jnp.sum(norm_costs * dist)

    best_cost = cost_of(0)
    best_idx = jnp.int32(0)
    for c in range(1, SAMPLES):
        cc = cost_of(c)
        better = cc < best_cost
        best_idx = jnp.where(better, jnp.int32(c), best_idx)
        best_cost = jnp.where(better, cc, best_cost)

    tops_a = jnp.stack(tops).astype(jnp.int32)
    lefts_a = jnp.stack(lefts).astype(jnp.int32)
    sel = jnp.zeros((16,), jnp.int32)
    sel = sel.at[0].set(tops_a[best_idx]).at[1].set(lefts_a[best_idx])

    best_image, best_label = _crop_kernel(image, label2d, sel)
    return best_image, best_label.reshape(1, CROP_H, CROP_W), best_cost


# speedup vs baseline: 3.7623x; 1.8579x over previous
import functools

import jax
import jax.numpy as jnp
from jax import lax
from jax.experimental import pallas as pl
from jax.experimental.pallas import tpu as pltpu
from jax.experimental.pallas import tpu_sc as plsc

H, W = 1024, 2048
CROP_H, CROP_W = 720, 1280
SAMPLES = 4
LABEL_COUNT = 19
NC, NS = 2, 16
NWORK = NC * NS
RPW = 23
WB = 1296
NVEC = WB // 16
NBANK = 4
HIST_W = LABEL_COUNT * 16 * NBANK

_mesh = plsc.VectorSubcoreMesh(core_axis_name="c", subcore_axis_name="s")
_params = pltpu.CompilerParams(
    use_tc_tiling_on_sc=False, needs_layout_passes=False)


_TOPS = (219, 196, 73, 29)
_LEFTS = (192, 367, 42, 696)


def _pick(vec, iota, k):
    return jnp.max(jnp.where(iota == k, vec, 0))


@functools.partial(
    pl.kernel,
    out_type=jax.ShapeDtypeStruct((NWORK, SAMPLES * HIST_W), jnp.int32),
    mesh=_mesh,
    scratch_types=[
        pltpu.VMEM((RPW, WB), jnp.int32),
        pltpu.VMEM((SAMPLES * HIST_W,), jnp.int32),
    ],
    compiler_params=_params,
)
def _hist_kernel(label_hbm, out_hbm, buf_v, hist_v):
    w = lax.axis_index("s") * NC + lax.axis_index("c")
    iota = lax.iota(jnp.int32, 16)
    zeros = jnp.zeros((16,), jnp.int32)
    ones = jnp.ones((16,), jnp.int32)

    for k in range(SAMPLES * HIST_W // 16):
        hist_v[pl.ds(k * 16, 16)] = zeros

    lo = jnp.minimum(RPW * w, CROP_H - RPW)
    r_begin = RPW * w

    for c in range(SAMPLES):
        top, left = _TOPS[c], _LEFTS[c]
        left_al = min(left & -16, W - WB)
        shift = left - left_al
        mask_first = iota >= shift
        mask_last = iota < shift

        pltpu.sync_copy(
            label_hbm.at[pl.ds(top + lo, RPW), left_al:left_al + WB], buf_v
        )

        def body(i, carry, _c=c, _mf=mask_first, _ml=mask_last):
            rg = lo + i
            valid = jnp.logical_and(rg >= r_begin, rg < CROP_H)
            rmask = jnp.full((16,), valid)
            for j in range(NVEC):
                lv = buf_v[i, pl.ds(j * 16, 16)]
                if j == 0:
                    m = jnp.logical_and(rmask, _mf)
                elif j == NVEC - 1:
                    m = jnp.logical_and(rmask, _ml)
                else:
                    m = rmask
                base = _c * HIST_W + (j % NBANK) * 16 + iota
                plsc.addupdate_scatter(
                    hist_v, [lv * (16 * NBANK) + base], ones, mask=m)
            return carry

        lax.fori_loop(0, RPW, body, 0)

    pltpu.sync_copy(hist_v, out_hbm.at[w])


@functools.partial(
    pl.kernel,
    out_type=(
        jax.ShapeDtypeStruct((3, CROP_H, CROP_W), jnp.float32),
        jax.ShapeDtypeStruct((CROP_H, CROP_W), jnp.int32),
    ),
    mesh=_mesh,
    scratch_types=[
        pltpu.VMEM((16,), jnp.int32),
        pltpu.VMEM((RPW, WB), jnp.float32),
        pltpu.VMEM((RPW, CROP_W), jnp.float32),
        pltpu.VMEM((RPW, WB), jnp.int32),
        pltpu.VMEM((RPW, CROP_W), jnp.int32),
    ],
    compiler_params=_params,
)
def _crop_kernel(img_hbm, lab_hbm, sel_hbm, oimg_hbm, olab_hbm,
                 sel_v, ibuf, obuf, lbuf, olbuf):
    w = lax.axis_index("s") * NC + lax.axis_index("c")
    iota = lax.iota(jnp.int32, 16)

    pltpu.sync_copy(sel_hbm, sel_v)
    sv = sel_v[...]
    top = _pick(sv, iota, 0)
    left = _pick(sv, iota, 1)
    left_al = pl.multiple_of(jnp.minimum(left & -16, W - WB), 16)
    shift = left - left_al
    lo = jnp.minimum(RPW * w, CROP_H - RPW)
    cbase = shift + iota

    def shift_rows(src, dst):
        def body(i, carry):
            rowv = jnp.full((16,), i)
            for j in range(CROP_W // 16):
                v = plsc.load_gather(src, [rowv, cbase + (j * 16)])
                dst[i, pl.ds(j * 16, 16)] = v
            return carry
        lax.fori_loop(0, RPW, body, 0)

    for ch in range(3):
        pltpu.sync_copy(
            img_hbm.at[ch, pl.ds(top + lo, RPW), pl.ds(left_al, WB)], ibuf)
        shift_rows(ibuf, obuf)
        pltpu.sync_copy(obuf, oimg_hbm.at[ch, pl.ds(lo, RPW), :])

    pltpu.sync_copy(
        lab_hbm.at[pl.ds(top + lo, RPW), pl.ds(left_al, WB)], lbuf)
    shift_rows(lbuf, olbuf)
    pltpu.sync_copy(olbuf, olab_hbm.at[pl.ds(lo, RPW), :])


def kernel(image, label_image, label_costs):
    label2d = label_image.reshape(H, W)

    parts = _hist_kernel(label2d)
    counts = parts.reshape(
        NWORK, SAMPLES, LABEL_COUNT, NBANK * 16).sum(axis=(0, 3))

    norm_costs = label_costs / jnp.maximum(jnp.sum(jnp.abs(label_costs)), 1e-12)
    total = float(CROP_H * CROP_W)

    def cost_of(c):
        dist = counts[c].astype(jnp.float32) / total
        return jnp.sum(norm_costs * dist)

    best_cost = cost_of(0)
    best_idx = jnp.int32(0)
    for c in range(1, SAMPLES):
        cc = cost_of(c)
        better = cc < best_cost
        best_idx = jnp.where(better, jnp.int32(c), best_idx)
        best_cost = jnp.where(better, cc, best_cost)

    tops_a = jnp.asarray(_TOPS, jnp.int32)
    lefts_a = jnp.asarray(_LEFTS, jnp.int32)
    sel = jnp.zeros((16,), jnp.int32)
    sel = sel.at[0].set(tops_a[best_idx]).at[1].set(lefts_a[best_idx])
    best_image, best_label = _crop_kernel(image, label2d, sel)
    return best_image, best_label.reshape(1, CROP_H, CROP_W), best_cost


# speedup vs baseline: 5.9873x; 1.5914x over previous
import functools

import jax
import jax.numpy as jnp
from jax import lax
from jax.experimental import pallas as pl
from jax.experimental.pallas import tpu as pltpu
from jax.experimental.pallas import tpu_sc as plsc

H, W = 1024, 2048
CROP_H, CROP_W = 720, 1280
SAMPLES = 4
LABEL_COUNT = 19
NC, NS = 2, 16
NWORK = NC * NS
RPW = 23
WB = 1296
NVEC = WB // 16
NBANK = 4
HIST_W = LABEL_COUNT * 16 * NBANK

_mesh = plsc.VectorSubcoreMesh(core_axis_name="c", subcore_axis_name="s")
_params = pltpu.CompilerParams(
    use_tc_tiling_on_sc=False, needs_layout_passes=False)


_TOPS = (219, 196, 73, 29)
_LEFTS = (192, 367, 42, 696)


def _pick(vec, iota, k):
    return jnp.max(jnp.where(iota == k, vec, 0))


@functools.partial(
    pl.kernel,
    out_type=jax.ShapeDtypeStruct((NWORK, SAMPLES * HIST_W), jnp.int32),
    mesh=_mesh,
    scratch_types=[
        pltpu.VMEM((RPW, WB), jnp.int32),
        pltpu.VMEM((SAMPLES * HIST_W,), jnp.int32),
    ],
    compiler_params=_params,
)
def _hist_kernel(label_hbm, out_hbm, buf_v, hist_v):
    w = lax.axis_index("s") * NC + lax.axis_index("c")
    iota = lax.iota(jnp.int32, 16)
    zeros = jnp.zeros((16,), jnp.int32)
    ones = jnp.ones((16,), jnp.int32)

    for k in range(SAMPLES * HIST_W // 16):
        hist_v[pl.ds(k * 16, 16)] = zeros

    lo = jnp.minimum(RPW * w, CROP_H - RPW)
    r_begin = RPW * w

    for c in range(SAMPLES):
        top, left = _TOPS[c], _LEFTS[c]
        left_al = min(left & -16, W - WB)
        shift = left - left_al
        mask_first = iota >= shift
        mask_last = iota < shift

        pltpu.sync_copy(
            label_hbm.at[pl.ds(top + lo, RPW), left_al:left_al + WB], buf_v
        )

        def body(i, carry, _c=c, _mf=mask_first, _ml=mask_last):
            rg = lo + i
            valid = jnp.logical_and(rg >= r_begin, rg < CROP_H)
            rmask = jnp.full((16,), valid)
            m_first = jnp.logical_and(rmask, _mf)
            m_last = jnp.logical_and(rmask, _ml)
            for g in range(0, NVEC, 8):
                js = range(g, min(g + 8, NVEC))
                idxs = []
                for j in js:
                    lv = buf_v[i, pl.ds(j * 16, 16)]
                    base = _c * HIST_W + (j % NBANK) * 16 + iota
                    idxs.append(lv * (16 * NBANK) + base)
                for k, j in enumerate(js):
                    m = m_first if j == 0 else (
                        m_last if j == NVEC - 1 else rmask)
                    plsc.addupdate_scatter(hist_v, [idxs[k]], ones, mask=m)
            return carry

        lax.fori_loop(0, RPW, body, 0)

    pltpu.sync_copy(hist_v, out_hbm.at[w])


@functools.partial(
    pl.kernel,
    out_type=(
        jax.ShapeDtypeStruct((3, CROP_H, CROP_W), jnp.float32),
        jax.ShapeDtypeStruct((CROP_H, CROP_W), jnp.int32),
    ),
    mesh=_mesh,
    scratch_types=[
        pltpu.VMEM((16,), jnp.int32),
        pltpu.VMEM((RPW, WB), jnp.float32),
        pltpu.VMEM((RPW, CROP_W), jnp.float32),
        pltpu.VMEM((RPW, WB), jnp.int32),
        pltpu.VMEM((RPW, CROP_W), jnp.int32),
    ],
    compiler_params=_params,
)
def _crop_kernel(img_hbm, lab_hbm, sel_hbm, oimg_hbm, olab_hbm,
                 sel_v, ibuf, obuf, lbuf, olbuf):
    w = lax.axis_index("s") * NC + lax.axis_index("c")
    iota = lax.iota(jnp.int32, 16)

    pltpu.sync_copy(sel_hbm, sel_v)
    sv = sel_v[...]
    top = _pick(sv, iota, 0)
    left = _pick(sv, iota, 1)
    left_al = pl.multiple_of(jnp.minimum(left & -16, W - WB), 16)
    shift = left - left_al
    lo = jnp.minimum(RPW * w, CROP_H - RPW)
    cbase = shift + iota

    def shift_rows(src, dst):
        def body(i, carry):
            rowv = jnp.full((16,), i)
            for g in range(0, CROP_W // 16, 8):
                js = range(g, min(g + 8, CROP_W // 16))
                vs = [plsc.load_gather(src, [rowv, cbase + (j * 16)])
                      for j in js]
                for k, j in enumerate(js):
                    dst[i, pl.ds(j * 16, 16)] = vs[k]
            return carry
        lax.fori_loop(0, RPW, body, 0)

    for ch in range(3):
        pltpu.sync_copy(
            img_hbm.at[ch, pl.ds(top + lo, RPW), pl.ds(left_al, WB)], ibuf)
        shift_rows(ibuf, obuf)
        pltpu.sync_copy(obuf, oimg_hbm.at[ch, pl.ds(lo, RPW), :])

    pltpu.sync_copy(
        lab_hbm.at[pl.ds(top + lo, RPW), pl.ds(left_al, WB)], lbuf)
    shift_rows(lbuf, olbuf)
    pltpu.sync_copy(olbuf, olab_hbm.at[pl.ds(lo, RPW), :])


def kernel(image, label_image, label_costs):
    label2d = label_image.reshape(H, W)

    parts = _hist_kernel(label2d)
    counts = parts.reshape(
        NWORK, SAMPLES, LABEL_COUNT, NBANK * 16).sum(axis=(0, 3))

    norm_costs = label_costs / jnp.maximum(jnp.sum(jnp.abs(label_costs)), 1e-12)
    total = float(CROP_H * CROP_W)

    def cost_of(c):
        dist = counts[c].astype(jnp.float32) / total
        return jnp.sum(norm_costs * dist)

    best_cost = cost_of(0)
    best_idx = jnp.int32(0)
    for c in range(1, SAMPLES):
        cc = cost_of(c)
        better = cc < best_cost
        best_idx = jnp.where(better, jnp.int32(c), best_idx)
        best_cost = jnp.where(better, cc, best_cost)

    tops_a = jnp.asarray(_TOPS, jnp.int32)
    lefts_a = jnp.asarray(_LEFTS, jnp.int32)
    sel = jnp.zeros((16,), jnp.int32)
    sel = sel.at[0].set(tops_a[best_idx]).at[1].set(lefts_a[best_idx])
    best_image, best_label = _crop_kernel(image, label2d, sel)
    return best_image, best_label.reshape(1, CROP_H, CROP_W), best_cost
